# baseline (speedup 1.0000x reference)
import functools

import jax
import jax.numpy as jnp
from jax import lax
from jax.experimental import pallas as pl
from jax.experimental.pallas import tpu as pltpu
from jax.experimental.pallas import tpu_sc as plsc

EMB = 64
HID = 256
BATCH = 16384

NC = 2
NS = 16
NW = NC * NS
CHUNK = 128
B_PER_W = BATCH // NW
K = B_PER_W // CHUNK
L = 16


def _gather_body(uids_hbm, iids_hbm, u2_hbm, i2_hbm, comb_hbm,
                 uids_v, iids_v, uidx_v, iidx_v, ustage_v, istage_v, comb_v,
                 semu, semi):
    wid = lax.axis_index("s") * NC + lax.axis_index("c")
    base = wid * B_PER_W
    for j in range(K):
        pltpu.sync_copy(uids_hbm.at[pl.ds(base + j * CHUNK, CHUNK)], uids_v.at[j])
        pltpu.sync_copy(iids_hbm.at[pl.ds(base + j * CHUNK, CHUNK)], iids_v.at[j])
    for j in range(K):
        for g in range(CHUNK // L):
            uidx_v[j, pl.ds(g * L, L)] = uids_v[j, pl.ds(g * L, L)] >> 1
            iidx_v[j, pl.ds(g * L, L)] = iids_v[j, pl.ds(g * L, L)] >> 1

    def fire(j):
        b = j % 2
        pltpu.async_copy(u2_hbm.at[uidx_v.at[j]], ustage_v.at[b], semu)
        pltpu.async_copy(i2_hbm.at[iidx_v.at[j]], istage_v.at[b], semi)

    def wait(j):
        b = j % 2
        pltpu.make_async_copy(u2_hbm.at[uidx_v.at[j]], ustage_v.at[b], semu).wait()
        pltpu.make_async_copy(i2_hbm.at[iidx_v.at[j]], istage_v.at[b], semi).wait()

    def extract(j):
        b = j % 2

        def group(t, _):
            uvec = uids_v[j, pl.ds(t * L, L)]
            ivec = iids_v[j, pl.ds(t * L, L)]
            for m in range(L):
                r = t * L + m
                uoff = (uvec[m] & 1) * EMB
                ioff = (ivec[m] & 1) * EMB
                for q in range(EMB // L):
                    comb_v[r, pl.ds(q * L, L)] = (
                        ustage_v[b, r, pl.ds(uoff + q * L, L)])
                    comb_v[r, pl.ds(EMB + q * L, L)] = (
                        istage_v[b, r, pl.ds(ioff + q * L, L)])
            return 0

        lax.fori_loop(0, CHUNK // L, group, 0)

    fire(0)
    for j in range(K):
        if j + 1 < K:
            fire(j + 1)
        wait(j)
        extract(j)
        pltpu.sync_copy(comb_v, comb_hbm.at[pl.ds(base + j * CHUNK, CHUNK)])


@jax.jit
def _gather(user_ids, item_ids, u2, i2):
    mesh = plsc.VectorSubcoreMesh(core_axis_name="c", subcore_axis_name="s")
    fn = functools.partial(
        pl.kernel,
        mesh=mesh,
        out_type=jax.ShapeDtypeStruct((BATCH, 2 * EMB), jnp.float32),
        scratch_types=[
            pltpu.VMEM((K, CHUNK), jnp.int32),
            pltpu.VMEM((K, CHUNK), jnp.int32),
            pltpu.VMEM((K, CHUNK), jnp.int32),
            pltpu.VMEM((K, CHUNK), jnp.int32),
            pltpu.VMEM((2, CHUNK, 2 * EMB), jnp.float32),
            pltpu.VMEM((2, CHUNK, 2 * EMB), jnp.float32),
            pltpu.VMEM((CHUNK, 2 * EMB), jnp.float32),
            pltpu.SemaphoreType.DMA,
            pltpu.SemaphoreType.DMA,
        ],
    )(_gather_body)
    return fn(user_ids, item_ids, u2, i2)


BS = 2048


def _mlp_body(c_ref, w1_ref, b1_ref, w2t_ref, b2_ref, out_ref):
    x = jnp.dot(c_ref[...], w1_ref[...], preferred_element_type=jnp.float32)
    x = jnp.maximum(x + b1_ref[...], 0.0)
    y = jnp.sum(x * w2t_ref[...], axis=1, keepdims=True)
    out_ref[...] = y + b2_ref[...]


@jax.jit
def _mlp(comb, w1, b1, w2t, b2):
    grid = (BATCH // BS,)
    return pl.pallas_call(
        _mlp_body,
        grid=grid,
        in_specs=[
            pl.BlockSpec((BS, 2 * EMB), lambda g: (g, 0)),
            pl.BlockSpec((2 * EMB, HID), lambda g: (0, 0)),
            pl.BlockSpec((1, HID), lambda g: (0, 0)),
            pl.BlockSpec((1, HID), lambda g: (0, 0)),
            pl.BlockSpec((1, 1), lambda g: (0, 0)),
        ],
        out_specs=pl.BlockSpec((BS, 1), lambda g: (g, 0)),
        out_shape=jax.ShapeDtypeStruct((BATCH, 1), jnp.float32),
    )(comb, w1, b1, w2t, b2)


def kernel(user_ids, item_ids, user_emb, item_emb, W1, b1, W2, b2):
    uids = user_ids.astype(jnp.int32)
    iids = item_ids.astype(jnp.int32)
    u2 = user_emb.reshape(-1, 2 * EMB)
    i2 = item_emb.reshape(-1, 2 * EMB)
    comb = _gather(uids, iids, u2, i2)
    return _mlp(comb, W1, b1.reshape(1, HID), W2.reshape(1, HID),
                b2.reshape(1, 1))


# speedup vs baseline: 1.2637x; 1.2637x over previous
import functools

import jax
import jax.numpy as jnp
from jax import lax
from jax.experimental import pallas as pl
from jax.experimental.pallas import tpu as pltpu
from jax.experimental.pallas import tpu_sc as plsc

EMB = 64
HID = 256
BATCH = 16384

NC = 2
NS = 16
NW = NC * NS
CHUNK = 128
B_PER_W = BATCH // NW
K = B_PER_W // CHUNK
L = 16


HU = 500736
HI = 50176


def _gather_body(uids_hbm, iids_hbm, u2_hbm, i2_hbm, comb_hbm,
                 uids_v, iids_v, uidx_v, iidx_v, ustage_v, istage_v, comb_v,
                 semu, semi):
    wid = lax.axis_index("s") * NC + lax.axis_index("c")
    base = wid * B_PER_W
    for j in range(K):
        pltpu.sync_copy(uids_hbm.at[pl.ds(base + j * CHUNK, CHUNK)], uids_v.at[j])
        pltpu.sync_copy(iids_hbm.at[pl.ds(base + j * CHUNK, CHUNK)], iids_v.at[j])
    for j in range(K):
        for g in range(CHUNK // L):
            uvec = uids_v[j, pl.ds(g * L, L)]
            ivec = iids_v[j, pl.ds(g * L, L)]
            uidx_v[j, pl.ds(g * L, L)] = uvec - jnp.where(uvec >= HU, HU, 0)
            iidx_v[j, pl.ds(g * L, L)] = ivec - jnp.where(ivec >= HI, HI, 0)

    def fire(j):
        b = j % 2
        pltpu.async_copy(u2_hbm.at[uidx_v.at[j]], ustage_v.at[b], semu)
        pltpu.async_copy(i2_hbm.at[iidx_v.at[j]], istage_v.at[b], semi)

    def wait(j):
        b = j % 2
        pltpu.make_async_copy(u2_hbm.at[uidx_v.at[j]], ustage_v.at[b], semu).wait()
        pltpu.make_async_copy(i2_hbm.at[iidx_v.at[j]], istage_v.at[b], semi).wait()

    def extract(j):
        b = j % 2

        def group(t, _):
            uvec = uids_v[j, pl.ds(t * L, L)]
            ivec = iids_v[j, pl.ds(t * L, L)]
            for m in range(L):
                r = t * L + m
                uoff = jnp.where(uvec[m] >= HU, EMB, 0)
                ioff = jnp.where(ivec[m] >= HI, EMB, 0)
                for q in range(EMB // L):
                    comb_v[r, pl.ds(q * L, L)] = (
                        ustage_v[b, r, pl.ds(uoff + q * L, L)])
                    comb_v[r, pl.ds(EMB + q * L, L)] = (
                        istage_v[b, r, pl.ds(ioff + q * L, L)])
            return 0

        lax.fori_loop(0, CHUNK // L, group, 0)

    fire(0)
    for j in range(K):
        if j + 1 < K:
            fire(j + 1)
        wait(j)
        extract(j)
        pltpu.sync_copy(comb_v, comb_hbm.at[pl.ds(base + j * CHUNK, CHUNK)])


@jax.jit
def _gather(user_ids, item_ids, u2, i2):
    mesh = plsc.VectorSubcoreMesh(core_axis_name="c", subcore_axis_name="s")
    fn = functools.partial(
        pl.kernel,
        mesh=mesh,
        out_type=jax.ShapeDtypeStruct((BATCH, 2 * EMB), jnp.float32),
        scratch_types=[
            pltpu.VMEM((K, CHUNK), jnp.int32),
            pltpu.VMEM((K, CHUNK), jnp.int32),
            pltpu.VMEM((K, CHUNK), jnp.int32),
            pltpu.VMEM((K, CHUNK), jnp.int32),
            pltpu.VMEM((2, CHUNK, 2 * EMB), jnp.float32),
            pltpu.VMEM((2, CHUNK, 2 * EMB), jnp.float32),
            pltpu.VMEM((CHUNK, 2 * EMB), jnp.float32),
            pltpu.SemaphoreType.DMA,
            pltpu.SemaphoreType.DMA,
        ],
    )(_gather_body)
    return fn(user_ids, item_ids, u2, i2)


def _tpose_body(a_ref, b_ref, out_ref):
    out_ref[:, 0:EMB] = jnp.swapaxes(a_ref[...], 0, 1)
    out_ref[:, EMB:2 * EMB] = jnp.swapaxes(b_ref[...], 0, 1)


TW = 1024


def _tpose(tab_t, h):
    nblk = h // TW
    bmax = tab_t.shape[1] // TW
    return pl.pallas_call(
        _tpose_body,
        grid=(nblk,),
        in_specs=[
            pl.BlockSpec((EMB, TW), lambda g: (0, g)),
            pl.BlockSpec((EMB, TW), lambda g: (0, jnp.minimum(g + nblk, bmax))),
        ],
        out_specs=pl.BlockSpec((TW, 2 * EMB), lambda g: (g, 0)),
        out_shape=jax.ShapeDtypeStruct((h, 2 * EMB), jnp.float32),
    )(tab_t, tab_t)


BS = 2048


def _mlp_body(c_ref, w1_ref, b1_ref, w2t_ref, b2_ref, out_ref):
    x = jnp.dot(c_ref[...], w1_ref[...], preferred_element_type=jnp.float32)
    x = jnp.maximum(x + b1_ref[...], 0.0)
    y = jnp.sum(x * w2t_ref[...], axis=1, keepdims=True)
    out_ref[...] = y + b2_ref[...]


@jax.jit
def _mlp(comb, w1, b1, w2t, b2):
    grid = (BATCH // BS,)
    return pl.pallas_call(
        _mlp_body,
        grid=grid,
        in_specs=[
            pl.BlockSpec((BS, 2 * EMB), lambda g: (g, 0)),
            pl.BlockSpec((2 * EMB, HID), lambda g: (0, 0)),
            pl.BlockSpec((1, HID), lambda g: (0, 0)),
            pl.BlockSpec((1, HID), lambda g: (0, 0)),
            pl.BlockSpec((1, 1), lambda g: (0, 0)),
        ],
        out_specs=pl.BlockSpec((BS, 1), lambda g: (g, 0)),
        out_shape=jax.ShapeDtypeStruct((BATCH, 1), jnp.float32),
    )(comb, w1, b1, w2t, b2)


def kernel(user_ids, item_ids, user_emb, item_emb, W1, b1, W2, b2):
    uids = user_ids.astype(jnp.int32)
    iids = item_ids.astype(jnp.int32)
    u2 = _tpose(user_emb.T, HU)
    i2 = _tpose(item_emb.T, HI)
    comb = _gather(uids, iids, u2, i2)
    return _mlp(comb, W1, b1.reshape(1, HID), W2.reshape(1, HID),
                b2.reshape(1, 1))


# speedup vs baseline: 2.0220x; 1.6001x over previous
import functools

import jax
import jax.numpy as jnp
from jax import lax
from jax.experimental import pallas as pl
from jax.experimental.pallas import tpu as pltpu
from jax.experimental.pallas import tpu_sc as plsc

EMB = 64
HID = 256
BATCH = 16384

NC = 2
NS = 16
NW = NC * NS
CHUNK = 128
B_PER_W = BATCH // NW
K = B_PER_W // CHUNK
L = 16


HU = 503808
HI = 53248


def _gather_body(uids_hbm, iids_hbm, u2_hbm, i2_hbm, comb_hbm,
                 uids_v, iids_v, uidx_v, iidx_v, ustage_v, istage_v, comb_v,
                 semu, semi):
    wid = lax.axis_index("s") * NC + lax.axis_index("c")
    base = wid * B_PER_W
    for j in range(K):
        pltpu.sync_copy(uids_hbm.at[pl.ds(base + j * CHUNK, CHUNK)], uids_v.at[j])
        pltpu.sync_copy(iids_hbm.at[pl.ds(base + j * CHUNK, CHUNK)], iids_v.at[j])
    for j in range(K):
        for g in range(CHUNK // L):
            uvec = uids_v[j, pl.ds(g * L, L)]
            ivec = iids_v[j, pl.ds(g * L, L)]
            uidx_v[j, pl.ds(g * L, L)] = uvec - jnp.where(uvec >= HU, HU, 0)
            iidx_v[j, pl.ds(g * L, L)] = ivec - jnp.where(ivec >= HI, HI, 0)

    def fire(j):
        b = j % 2
        pltpu.async_copy(u2_hbm.at[uidx_v.at[j]], ustage_v.at[b], semu)
        pltpu.async_copy(i2_hbm.at[iidx_v.at[j]], istage_v.at[b], semi)

    def wait(j):
        b = j % 2
        pltpu.make_async_copy(u2_hbm.at[uidx_v.at[j]], ustage_v.at[b], semu).wait()
        pltpu.make_async_copy(i2_hbm.at[iidx_v.at[j]], istage_v.at[b], semi).wait()

    def extract(j):
        b = j % 2

        def group(t, _):
            uvec = uids_v[j, pl.ds(t * L, L)]
            ivec = iids_v[j, pl.ds(t * L, L)]
            for m in range(L):
                r = t * L + m
                uoff = jnp.where(uvec[m] >= HU, EMB, 0)
                ioff = jnp.where(ivec[m] >= HI, EMB, 0)
                for q in range(EMB // L):
                    comb_v[r, pl.ds(q * L, L)] = (
                        ustage_v[b, r, pl.ds(uoff + q * L, L)])
                    comb_v[r, pl.ds(EMB + q * L, L)] = (
                        istage_v[b, r, pl.ds(ioff + q * L, L)])
            return 0

        lax.fori_loop(0, CHUNK // L, group, 0)

    fire(0)
    for j in range(K):
        if j + 1 < K:
            fire(j + 1)
        wait(j)
        extract(j)
        pltpu.sync_copy(comb_v, comb_hbm.at[pl.ds(base + j * CHUNK, CHUNK)])


@jax.jit
def _gather(user_ids, item_ids, u2, i2):
    mesh = plsc.VectorSubcoreMesh(core_axis_name="c", subcore_axis_name="s")
    fn = functools.partial(
        pl.kernel,
        mesh=mesh,
        out_type=jax.ShapeDtypeStruct((BATCH, 2 * EMB), jnp.float32),
        scratch_types=[
            pltpu.VMEM((K, CHUNK), jnp.int32),
            pltpu.VMEM((K, CHUNK), jnp.int32),
            pltpu.VMEM((K, CHUNK), jnp.int32),
            pltpu.VMEM((K, CHUNK), jnp.int32),
            pltpu.VMEM((2, CHUNK, 2 * EMB), jnp.float32),
            pltpu.VMEM((2, CHUNK, 2 * EMB), jnp.float32),
            pltpu.VMEM((CHUNK, 2 * EMB), jnp.float32),
            pltpu.SemaphoreType.DMA,
            pltpu.SemaphoreType.DMA,
        ],
    )(_gather_body)
    return fn(user_ids, item_ids, u2, i2)


def _tpose_body(a_ref, b_ref, out_ref):
    out_ref[:, 0:EMB] = jnp.swapaxes(a_ref[...], 0, 1)
    out_ref[:, EMB:2 * EMB] = jnp.swapaxes(b_ref[...], 0, 1)


TW = 4096


def _tpose(tab_t, h):
    nblk = h // TW
    bmax = tab_t.shape[1] // TW
    return pl.pallas_call(
        _tpose_body,
        grid=(nblk,),
        in_specs=[
            pl.BlockSpec((EMB, TW), lambda g: (0, g)),
            pl.BlockSpec((EMB, TW), lambda g: (0, jnp.minimum(g + nblk, bmax))),
        ],
        out_specs=pl.BlockSpec((TW, 2 * EMB), lambda g: (g, 0)),
        out_shape=jax.ShapeDtypeStruct((h, 2 * EMB), jnp.float32),
    )(tab_t, tab_t)


BS = 2048


def _mlp_body(c_ref, w1_ref, b1_ref, w2t_ref, b2_ref, out_ref):
    x = jnp.dot(c_ref[...], w1_ref[...], preferred_element_type=jnp.float32)
    x = jnp.maximum(x + b1_ref[...], 0.0)
    y = jnp.sum(x * w2t_ref[...], axis=1, keepdims=True)
    out_ref[...] = y + b2_ref[...]


@jax.jit
def _mlp(comb, w1, b1, w2t, b2):
    grid = (BATCH // BS,)
    return pl.pallas_call(
        _mlp_body,
        grid=grid,
        in_specs=[
            pl.BlockSpec((BS, 2 * EMB), lambda g: (g, 0)),
            pl.BlockSpec((2 * EMB, HID), lambda g: (0, 0)),
            pl.BlockSpec((1, HID), lambda g: (0, 0)),
            pl.BlockSpec((1, HID), lambda g: (0, 0)),
            pl.BlockSpec((1, 1), lambda g: (0, 0)),
        ],
        out_specs=pl.BlockSpec((BS, 1), lambda g: (g, 0)),
        out_shape=jax.ShapeDtypeStruct((BATCH, 1), jnp.float32),
    )(comb, w1, b1, w2t, b2)


def kernel(user_ids, item_ids, user_emb, item_emb, W1, b1, W2, b2):
    uids = user_ids.astype(jnp.int32)
    iids = item_ids.astype(jnp.int32)
    u2 = _tpose(user_emb.T, HU)
    i2 = _tpose(item_emb.T, HI)
    comb = _gather(uids, iids, u2, i2)
    return _mlp(comb, W1, b1.reshape(1, HID), W2.reshape(1, HID),
                b2.reshape(1, 1))


# speedup vs baseline: 2.2431x; 1.1094x over previous
import functools

import jax
import jax.numpy as jnp
from jax import lax
from jax.experimental import pallas as pl
from jax.experimental.pallas import tpu as pltpu
from jax.experimental.pallas import tpu_sc as plsc

EMB = 64
HID = 256
BATCH = 16384

NC = 2
NS = 16
NW = NC * NS
CHUNK = 128
B_PER_W = BATCH // NW
K = B_PER_W // CHUNK
L = 16


HU = 507904
HI = 57344


def _gather_body(uids_hbm, iids_hbm, u2_hbm, i2_hbm, comb_hbm,
                 uids_v, iids_v, uidx_v, iidx_v, ustage_v, istage_v, comb_v,
                 semu, semi):
    wid = lax.axis_index("s") * NC + lax.axis_index("c")
    base = wid * B_PER_W
    for j in range(K):
        pltpu.sync_copy(uids_hbm.at[pl.ds(base + j * CHUNK, CHUNK)], uids_v.at[j])
        pltpu.sync_copy(iids_hbm.at[pl.ds(base + j * CHUNK, CHUNK)], iids_v.at[j])
    for j in range(K):
        for g in range(CHUNK // L):
            uvec = uids_v[j, pl.ds(g * L, L)]
            ivec = iids_v[j, pl.ds(g * L, L)]
            uidx_v[j, pl.ds(g * L, L)] = uvec - jnp.where(uvec >= HU, HU, 0)
            iidx_v[j, pl.ds(g * L, L)] = ivec - jnp.where(ivec >= HI, HI, 0)

    def fire(j):
        b = j % 2
        pltpu.async_copy(u2_hbm.at[uidx_v.at[j]], ustage_v.at[b], semu)
        pltpu.async_copy(i2_hbm.at[iidx_v.at[j]], istage_v.at[b], semi)

    def wait(j):
        b = j % 2
        pltpu.make_async_copy(u2_hbm.at[uidx_v.at[j]], ustage_v.at[b], semu).wait()
        pltpu.make_async_copy(i2_hbm.at[iidx_v.at[j]], istage_v.at[b], semi).wait()

    def extract(j):
        b = j % 2

        def group(t, _):
            uvec = uids_v[j, pl.ds(t * L, L)]
            ivec = iids_v[j, pl.ds(t * L, L)]
            for m in range(L):
                r = t * L + m
                uoff = jnp.where(uvec[m] >= HU, EMB, 0)
                ioff = jnp.where(ivec[m] >= HI, EMB, 0)
                for q in range(EMB // L):
                    comb_v[r, pl.ds(q * L, L)] = (
                        ustage_v[b, r, pl.ds(uoff + q * L, L)])
                    comb_v[r, pl.ds(EMB + q * L, L)] = (
                        istage_v[b, r, pl.ds(ioff + q * L, L)])
            return 0

        lax.fori_loop(0, CHUNK // L, group, 0)

    fire(0)
    for j in range(K):
        if j + 1 < K:
            fire(j + 1)
        wait(j)
        extract(j)
        pltpu.sync_copy(comb_v, comb_hbm.at[pl.ds(base + j * CHUNK, CHUNK)])


@jax.jit
def _gather(user_ids, item_ids, u2, i2):
    mesh = plsc.VectorSubcoreMesh(core_axis_name="c", subcore_axis_name="s")
    fn = functools.partial(
        pl.kernel,
        mesh=mesh,
        out_type=jax.ShapeDtypeStruct((BATCH, 2 * EMB), jnp.float32),
        scratch_types=[
            pltpu.VMEM((K, CHUNK), jnp.int32),
            pltpu.VMEM((K, CHUNK), jnp.int32),
            pltpu.VMEM((K, CHUNK), jnp.int32),
            pltpu.VMEM((K, CHUNK), jnp.int32),
            pltpu.VMEM((2, CHUNK, 2 * EMB), jnp.float32),
            pltpu.VMEM((2, CHUNK, 2 * EMB), jnp.float32),
            pltpu.VMEM((CHUNK, 2 * EMB), jnp.float32),
            pltpu.SemaphoreType.DMA,
            pltpu.SemaphoreType.DMA,
        ],
    )(_gather_body)
    return fn(user_ids, item_ids, u2, i2)


def _tpose_body(a_ref, b_ref, out_ref):
    out_ref[:, 0:EMB] = jnp.swapaxes(a_ref[...], 0, 1)
    out_ref[:, EMB:2 * EMB] = jnp.swapaxes(b_ref[...], 0, 1)


TW = 8192


def _tpose(tab_t, h):
    nblk = h // TW
    bmax = tab_t.shape[1] // TW
    return pl.pallas_call(
        _tpose_body,
        grid=(nblk,),
        in_specs=[
            pl.BlockSpec((EMB, TW), lambda g: (0, g)),
            pl.BlockSpec((EMB, TW), lambda g: (0, jnp.minimum(g + nblk, bmax))),
        ],
        out_specs=pl.BlockSpec((TW, 2 * EMB), lambda g: (g, 0)),
        out_shape=jax.ShapeDtypeStruct((h, 2 * EMB), jnp.float32),
    )(tab_t, tab_t)


BS = 2048


def _mlp_body(c_ref, w1_ref, b1_ref, w2t_ref, b2_ref, out_ref):
    x = jnp.dot(c_ref[...], w1_ref[...], preferred_element_type=jnp.float32)
    x = jnp.maximum(x + b1_ref[...], 0.0)
    y = jnp.sum(x * w2t_ref[...], axis=1, keepdims=True)
    out_ref[...] = y + b2_ref[...]


@jax.jit
def _mlp(comb, w1, b1, w2t, b2):
    grid = (BATCH // BS,)
    return pl.pallas_call(
        _mlp_body,
        grid=grid,
        in_specs=[
            pl.BlockSpec((BS, 2 * EMB), lambda g: (g, 0)),
            pl.BlockSpec((2 * EMB, HID), lambda g: (0, 0)),
            pl.BlockSpec((1, HID), lambda g: (0, 0)),
            pl.BlockSpec((1, HID), lambda g: (0, 0)),
            pl.BlockSpec((1, 1), lambda g: (0, 0)),
        ],
        out_specs=pl.BlockSpec((BS, 1), lambda g: (g, 0)),
        out_shape=jax.ShapeDtypeStruct((BATCH, 1), jnp.float32),
    )(comb, w1, b1, w2t, b2)


def kernel(user_ids, item_ids, user_emb, item_emb, W1, b1, W2, b2):
    uids = user_ids.astype(jnp.int32)
    iids = item_ids.astype(jnp.int32)
    u2 = _tpose(user_emb.T, HU)
    i2 = _tpose(item_emb.T, HI)
    comb = _gather(uids, iids, u2, i2)
    return _mlp(comb, W1, b1.reshape(1, HID), W2.reshape(1, HID),
                b2.reshape(1, 1))
